# baseline (speedup 1.0000x reference)
import functools

import jax
import jax.numpy as jnp
from jax import lax
from jax.experimental import pallas as pl
from jax.experimental.pallas import tpu as pltpu
from jax.experimental.pallas import tpu_sc as plsc

NUM_CORES = 2
NUM_SUBCORES = 16
NUM_WORKERS = NUM_CORES * NUM_SUBCORES

BATCH = 16384
HIST_LEN = 20
FEATURES = 64
B = BATCH * HIST_LEN
B_PER_W = B // NUM_WORKERS
CHUNK = 128
NCHUNK = B_PER_W // CHUNK


def _embed_kernel(table_hbm, idx_hbm, out_hbm, idx_v, rows_v, gsem):
    wid = lax.axis_index("s") * NUM_CORES + lax.axis_index("c")
    base = wid * B_PER_W

    pltpu.sync_copy(idx_hbm.at[pl.ds(base, B_PER_W)], idx_v)

    def start_gather(i, buf):
        pltpu.async_copy(
            table_hbm.at[idx_v.at[pl.ds(i * CHUNK, CHUNK)]],
            rows_v.at[buf],
            gsem,
        )

    def finish_and_store(i, buf):
        pltpu.make_async_copy(
            table_hbm.at[idx_v.at[pl.ds(i * CHUNK, CHUNK)]],
            rows_v.at[buf],
            gsem,
        ).wait()
        pltpu.sync_copy(rows_v.at[buf], out_hbm.at[pl.ds(base + i * CHUNK, CHUNK)])

    start_gather(0, 0)

    @pl.loop(0, NCHUNK, step=2)
    def _(i):
        start_gather(i + 1, 1)
        finish_and_store(i, 0)
        @pl.when(i + 2 < NCHUNK)
        def _():
            start_gather(i + 2, 0)
        finish_and_store(i + 1, 1)


@jax.jit
def kernel(inputs, embedding):
    idx_flat = inputs.reshape(-1).astype(jnp.int32)
    mesh = plsc.VectorSubcoreMesh(
        core_axis_name="c", subcore_axis_name="s",
        num_cores=NUM_CORES, num_subcores=NUM_SUBCORES,
    )
    run = pl.kernel(
        _embed_kernel,
        out_type=jax.ShapeDtypeStruct((B, FEATURES), jnp.float32),
        mesh=mesh,
        scratch_types=[
            pltpu.VMEM((B_PER_W,), jnp.int32),
            pltpu.VMEM((2, CHUNK, FEATURES), jnp.float32),
            pltpu.SemaphoreType.DMA,
        ],
        compiler_params=pltpu.CompilerParams(use_tc_tiling_on_sc=False),
    )
    out = run(embedding, idx_flat)
    return out.reshape(BATCH, HIST_LEN, FEATURES)


# speedup vs baseline: 1.0119x; 1.0119x over previous
import functools

import jax
import jax.numpy as jnp
from jax import lax
from jax.experimental import pallas as pl
from jax.experimental.pallas import tpu as pltpu
from jax.experimental.pallas import tpu_sc as plsc

NUM_CORES = 2
NUM_SUBCORES = 16
NUM_WORKERS = NUM_CORES * NUM_SUBCORES

BATCH = 16384
HIST_LEN = 20
FEATURES = 64
B = BATCH * HIST_LEN
B_PER_W = B // NUM_WORKERS
CHUNK = 512
NCHUNK = B_PER_W // CHUNK


def _embed_kernel(table_hbm, idx_hbm, out_hbm, idx_v, rows_v, gsem):
    wid = lax.axis_index("s") * NUM_CORES + lax.axis_index("c")
    base = wid * B_PER_W

    pltpu.sync_copy(idx_hbm.at[pl.ds(base, B_PER_W)], idx_v)

    def start_gather(i, buf):
        pltpu.async_copy(
            table_hbm.at[idx_v.at[pl.ds(i * CHUNK, CHUNK)]],
            rows_v.at[buf],
            gsem,
        )

    def finish_and_store(i, buf):
        pltpu.make_async_copy(
            table_hbm.at[idx_v.at[pl.ds(i * CHUNK, CHUNK)]],
            rows_v.at[buf],
            gsem,
        ).wait()
        pltpu.sync_copy(rows_v.at[buf], out_hbm.at[pl.ds(base + i * CHUNK, CHUNK)])

    start_gather(0, 0)

    @pl.loop(0, NCHUNK, step=2)
    def _(i):
        start_gather(i + 1, 1)
        finish_and_store(i, 0)
        @pl.when(i + 2 < NCHUNK)
        def _():
            start_gather(i + 2, 0)
        finish_and_store(i + 1, 1)


@jax.jit
def kernel(inputs, embedding):
    idx_flat = inputs.reshape(-1).astype(jnp.int32)
    mesh = plsc.VectorSubcoreMesh(
        core_axis_name="c", subcore_axis_name="s",
        num_cores=NUM_CORES, num_subcores=NUM_SUBCORES,
    )
    run = pl.kernel(
        _embed_kernel,
        out_type=jax.ShapeDtypeStruct((B, FEATURES), jnp.float32),
        mesh=mesh,
        scratch_types=[
            pltpu.VMEM((B_PER_W,), jnp.int32),
            pltpu.VMEM((2, CHUNK, FEATURES), jnp.float32),
            pltpu.SemaphoreType.DMA,
        ],
        compiler_params=pltpu.CompilerParams(use_tc_tiling_on_sc=False),
    )
    out = run(embedding, idx_flat)
    return out.reshape(BATCH, HIST_LEN, FEATURES)


# speedup vs baseline: 1.0529x; 1.0405x over previous
import functools

import jax
import jax.numpy as jnp
from jax import lax
from jax.experimental import pallas as pl
from jax.experimental.pallas import tpu as pltpu
from jax.experimental.pallas import tpu_sc as plsc

NUM_CORES = 2
NUM_SUBCORES = 16
NUM_WORKERS = NUM_CORES * NUM_SUBCORES

BATCH = 16384
HIST_LEN = 20
FEATURES = 64
B = BATCH * HIST_LEN
B_PER_W = B // NUM_WORKERS
CHUNK = 512
NCHUNK = B_PER_W // CHUNK


def _embed_kernel(table_hbm, idx_hbm, out_hbm, idx_v, rows_v, gsem):
    wid = lax.axis_index("s") * NUM_CORES + lax.axis_index("c")
    base = wid * B_PER_W

    pltpu.sync_copy(idx_hbm.at[pl.ds(base, B_PER_W)], idx_v)

    def start_gather(i, buf):
        pltpu.async_copy(
            table_hbm.at[idx_v.at[pl.ds(i * CHUNK, CHUNK)]],
            rows_v.at[buf],
            gsem,
        )

    def finish_and_store(i, buf):
        pltpu.make_async_copy(
            table_hbm.at[idx_v.at[pl.ds(i * CHUNK, CHUNK)]],
            rows_v.at[buf],
            gsem,
        ).wait()
        pltpu.sync_copy(rows_v.at[buf], out_hbm.at[pl.ds(base + i * CHUNK, CHUNK)])

    start_gather(0, 0)

    @pl.loop(0, NCHUNK, step=2)
    def _(i):
        start_gather(i + 1, 1)
        finish_and_store(i, 0)
        @pl.when(i + 2 < NCHUNK)
        def _():
            start_gather(i + 2, 0)
        finish_and_store(i + 1, 1)


@jax.jit
def kernel(inputs, embedding):
    idx_flat = inputs.T.reshape(-1).astype(jnp.int32)
    mesh = plsc.VectorSubcoreMesh(
        core_axis_name="c", subcore_axis_name="s",
        num_cores=NUM_CORES, num_subcores=NUM_SUBCORES,
    )
    run = pl.kernel(
        _embed_kernel,
        out_type=jax.ShapeDtypeStruct((B, FEATURES), jnp.float32),
        mesh=mesh,
        scratch_types=[
            pltpu.VMEM((B_PER_W,), jnp.int32),
            pltpu.VMEM((2, CHUNK, FEATURES), jnp.float32),
            pltpu.SemaphoreType.DMA,
        ],
        compiler_params=pltpu.CompilerParams(use_tc_tiling_on_sc=False),
    )
    out = run(embedding, idx_flat)
    return out.reshape(HIST_LEN, BATCH, FEATURES).transpose(1, 0, 2)
